# baseline (speedup 1.0000x reference)
import functools

import jax
import jax.numpy as jnp
from jax import lax
from jax.experimental import pallas as pl
from jax.experimental.pallas import tpu as pltpu
from jax.experimental.pallas import tpu_sc as plsc

NC = 2
NS = 16
NW = NC * NS
K = 768
KD = 1024
BR = 2000


def _mesh():
    return plsc.VectorSubcoreMesh(
        core_axis_name="c", subcore_axis_name="s", num_cores=NC, num_subcores=NS)


def _zero_acc(rows, acc, zbase, rpt, kk):
    @pl.loop(0, kk)
    def _(i):
        rows[i, :] = jnp.zeros((16,), jnp.float32)
    nfull, rem = rpt // kk, rpt % kk
    for j in range(nfull):
        pltpu.sync_copy(rows, acc.at[pl.ds(zbase + j * kk, kk)])
    if rem:
        pltpu.sync_copy(rows.at[pl.ds(0, rem)],
                        acc.at[pl.ds(zbase + nfull * kk, rem)])


def _write_out(acc, out_hbm, obase, zbase, rpt, kk):
    nfull, rem = rpt // kk, rpt % kk
    for j in range(nfull):
        pltpu.sync_copy(acc.at[pl.ds(zbase + j * kk, kk)],
                        out_hbm.at[pl.ds(obase + j * kk, kk)])
    if rem:
        pltpu.sync_copy(acc.at[pl.ds(zbase + nfull * kk, rem)],
                        out_hbm.at[pl.ds(obase + nfull * kk, rem)])


def _make_edge_kernel(A, D, n_chunks):
    rpt = A // NS
    assert n_chunks % 2 == 0

    @functools.partial(
        pl.kernel,
        out_type=jax.ShapeDtypeStruct((NC * A, D), jnp.float32),
        mesh=_mesh(),
        compiler_params=pltpu.CompilerParams(use_tc_tiling_on_sc=False),
        scratch_types=[
            pltpu.VMEM((K,), jnp.int32),
            pltpu.VMEM((K,), jnp.int32),
            pltpu.VMEM((K,), jnp.int32),
            pltpu.VMEM((K,), jnp.int32),
            pltpu.VMEM((K, D), jnp.float32),
            pltpu.VMEM((K, D), jnp.float32),
            pltpu.VMEM_SHARED((A, D), jnp.float32),
            pltpu.SemaphoreType.DMA,
            pltpu.SemaphoreType.DMA,
        ],
    )
    def k(proj_hbm, src_hbm, dst_hbm, out_hbm,
          sidx0, sidx1, didx0, didx1, rows0, rows1, acc, sem0, sem1):
        c = lax.axis_index("c")
        s = lax.axis_index("s")
        wid = s * NC + c
        zbase = s * rpt
        _zero_acc(rows0, acc, zbase, rpt, K)
        plsc.subcore_barrier()
        ebase = wid * (n_chunks * K)
        sidx = (sidx0, sidx1)
        didx = (didx0, didx1)
        rows = (rows0, rows1)
        sem = (sem0, sem1)

        pltpu.sync_copy(src_hbm.at[pl.ds(ebase, K)], sidx0)
        pltpu.sync_copy(dst_hbm.at[pl.ds(ebase, K)], didx0)
        pltpu.async_copy(proj_hbm.at[sidx0], rows0, sem0)

        @pl.loop(0, n_chunks // 2)
        def _(t):
            for b in (0, 1):
                i = 2 * t + b
                nb = 1 - b
                nxt = jnp.minimum(i + 1, n_chunks - 1)
                noff = ebase + nxt * K
                pltpu.sync_copy(src_hbm.at[pl.ds(noff, K)], sidx[nb])
                pltpu.sync_copy(dst_hbm.at[pl.ds(noff, K)], didx[nb])
                pltpu.async_copy(proj_hbm.at[sidx[nb]], rows[nb], sem[nb])
                pltpu.make_async_copy(proj_hbm.at[sidx[b]], rows[b],
                                      sem[b]).wait()
                pltpu.sync_copy(rows[b], acc.at[didx[b]], add=True)

        pltpu.make_async_copy(proj_hbm.at[sidx0], rows0, sem0).wait()
        plsc.subcore_barrier()
        _write_out(acc, out_hbm, c * A + zbase, zbase, rpt, K)

    return k


def _make_deg_kernel(A, D, n_chunks):
    rpt = A // NS

    @functools.partial(
        pl.kernel,
        out_type=jax.ShapeDtypeStruct((NC * A, D), jnp.float32),
        mesh=_mesh(),
        compiler_params=pltpu.CompilerParams(use_tc_tiling_on_sc=False),
        scratch_types=[
            pltpu.VMEM((KD,), jnp.int32),
            pltpu.VMEM((KD, D), jnp.float32),
            pltpu.VMEM_SHARED((A, D), jnp.float32),
        ],
    )
    def k(dst_hbm, out_hbm, didx, rows, acc):
        c = lax.axis_index("c")
        s = lax.axis_index("s")
        wid = s * NC + c
        zbase = s * rpt
        _zero_acc(rows, acc, zbase, rpt, KD)
        plsc.subcore_barrier()

        @pl.loop(0, KD)
        def _(i):
            rows[i, :] = jnp.ones((16,), jnp.float32)

        ebase = wid * (n_chunks * KD)

        @pl.loop(0, n_chunks)
        def _(i):
            off = ebase + i * KD
            pltpu.sync_copy(dst_hbm.at[pl.ds(off, KD)], didx)
            pltpu.sync_copy(rows, acc.at[didx], add=True)

        plsc.subcore_barrier()
        _write_out(acc, out_hbm, c * A + zbase, zbase, rpt, KD)

    return k


def _mm0_body(x_ref, wm_ref, wf_ref, b_ref, proj_ref, film_ref):
    h = x_ref[...]
    proj_ref[...] = jnp.dot(h, wm_ref[...], preferred_element_type=jnp.float32)
    film_ref[...] = (jnp.dot(h, wf_ref[...], preferred_element_type=jnp.float32)
                     + b_ref[0:1, :])


def _combine(s0_ref, s1_ref, f_ref, d0_ref, d1_ref, D):
    Ssum = s0_ref[0] + s1_ref[0]
    deg = d0_ref[0] + d1_ref[0]
    gamma = f_ref[:, :D]
    beta = f_ref[:, D:]
    return jnp.maximum(gamma * Ssum + deg * beta, 0.0)


def _make_mm0(N, D):
    grid = N // BR
    return pl.pallas_call(
        _mm0_body,
        grid=(grid,),
        in_specs=[
            pl.BlockSpec((BR, D), lambda i: (i, 0)),
            pl.BlockSpec((D, D), lambda i: (0, 0)),
            pl.BlockSpec((D, 2 * D), lambda i: (0, 0)),
            pl.BlockSpec((8, 2 * D), lambda i: (0, 0)),
        ],
        out_specs=[
            pl.BlockSpec((BR, D), lambda i: (i, 0)),
            pl.BlockSpec((BR, 2 * D), lambda i: (i, 0)),
        ],
        out_shape=[
            jax.ShapeDtypeStruct((N, D), jnp.float32),
            jax.ShapeDtypeStruct((N, 2 * D), jnp.float32),
        ],
    )


def _make_mmc(N, A, D):
    grid = N // BR
    nblk_a = A // BR

    def body(s_ref, s1_ref, f_ref, d0_ref, d1_ref, wm_ref, wf_ref, b_ref,
             proj_ref, film_ref):
        h = _combine(s_ref, s1_ref, f_ref, d0_ref, d1_ref, D)
        proj_ref[...] = jnp.dot(h, wm_ref[...],
                                preferred_element_type=jnp.float32)
        film_ref[...] = (jnp.dot(h, wf_ref[...],
                                 preferred_element_type=jnp.float32)
                         + b_ref[0:1, :])

    sp = pl.BlockSpec((1, BR, D), lambda i: (0, i, 0))
    return pl.pallas_call(
        body,
        grid=(grid,),
        in_specs=[
            sp,
            pl.BlockSpec((1, BR, D), lambda i: (1, i, 0)),
            pl.BlockSpec((BR, 2 * D), lambda i: (i, 0)),
            sp,
            pl.BlockSpec((1, BR, D), lambda i: (1, i, 0)),
            pl.BlockSpec((D, D), lambda i: (0, 0)),
            pl.BlockSpec((D, 2 * D), lambda i: (0, 0)),
            pl.BlockSpec((8, 2 * D), lambda i: (0, 0)),
        ],
        out_specs=[
            pl.BlockSpec((BR, D), lambda i: (i, 0)),
            pl.BlockSpec((BR, 2 * D), lambda i: (i, 0)),
        ],
        out_shape=[
            jax.ShapeDtypeStruct((N, D), jnp.float32),
            jax.ShapeDtypeStruct((N, 2 * D), jnp.float32),
        ],
    )


def _make_mmf(N, A, D):
    grid = N // BR

    def body(s_ref, s1_ref, f_ref, d0_ref, d1_ref, h_ref):
        h_ref[...] = _combine(s_ref, s1_ref, f_ref, d0_ref, d1_ref, D)

    return pl.pallas_call(
        body,
        grid=(grid,),
        in_specs=[
            pl.BlockSpec((1, BR, D), lambda i: (0, i, 0)),
            pl.BlockSpec((1, BR, D), lambda i: (1, i, 0)),
            pl.BlockSpec((BR, 2 * D), lambda i: (i, 0)),
            pl.BlockSpec((1, BR, D), lambda i: (0, i, 0)),
            pl.BlockSpec((1, BR, D), lambda i: (1, i, 0)),
        ],
        out_specs=[pl.BlockSpec((BR, D), lambda i: (i, 0))],
        out_shape=[jax.ShapeDtypeStruct((N, D), jnp.float32)],
    )


def kernel(x, edge_index, W_msg, W_film, b_film):
    N, D = x.shape
    E = edge_index.shape[1]
    L = W_msg.shape[0]
    assert D == 16

    A = -(-(N + 1) // NS) * NS
    n_ch_e = -(-E // (NW * K))
    n_ch_e += n_ch_e % 2
    n_ch_d = -(-E // (NW * KD))
    E_pad = max(NW * K * n_ch_e, NW * KD * n_ch_d)
    pad = E_pad - E

    src = edge_index[0]
    dst = edge_index[1]
    if pad:
        src = jnp.concatenate([src, jnp.zeros((pad,), jnp.int32)])
        dst = jnp.concatenate([dst, jnp.full((pad,), N, jnp.int32)])

    edge_k = _make_edge_kernel(A, D, n_ch_e)
    deg_k = _make_deg_kernel(A, D, n_ch_d)
    mm0 = _make_mm0(N, D)
    mmc = _make_mmc(N, A, D)
    mmf = _make_mmf(N, A, D)

    b2 = jnp.broadcast_to(b_film[:, None, :], (L, 8, 2 * D))

    degp = deg_k(dst).reshape(NC, A, D)
    proj, film = mm0(x, W_msg[0], W_film[0], b2[0])
    for l in range(L):
        Sp = edge_k(proj, src, dst).reshape(NC, A, D)
        if l < L - 1:
            proj, film = mmc(Sp, Sp, film, degp, degp,
                             W_msg[l + 1], W_film[l + 1], b2[l + 1])
        else:
            (h,) = mmf(Sp, Sp, film, degp, degp)
    return h


# speedup vs baseline: 1.1458x; 1.1458x over previous
import functools

import jax
import jax.numpy as jnp
from jax import lax
from jax.experimental import pallas as pl
from jax.experimental.pallas import tpu as pltpu
from jax.experimental.pallas import tpu_sc as plsc

NC = 2
NS = 16
NW = NC * NS
K = 1000
BR = 2000


def _mesh():
    return plsc.VectorSubcoreMesh(
        core_axis_name="c", subcore_axis_name="s", num_cores=NC, num_subcores=NS)


def _zero_acc(rows, acc, zbase, rpt, kk):
    @pl.loop(0, kk)
    def _(i):
        rows[i, :] = jnp.zeros((16,), jnp.float32)
    nfull, rem = rpt // kk, rpt % kk
    for j in range(nfull):
        pltpu.sync_copy(rows, acc.at[pl.ds(zbase + j * kk, kk)])
    if rem:
        pltpu.sync_copy(rows.at[pl.ds(0, rem)],
                        acc.at[pl.ds(zbase + nfull * kk, rem)])


def _write_out(acc, out_hbm, obase, zbase, rpt, kk):
    nfull, rem = rpt // kk, rpt % kk
    for j in range(nfull):
        pltpu.sync_copy(acc.at[pl.ds(zbase + j * kk, kk)],
                        out_hbm.at[pl.ds(obase + j * kk, kk)])
    if rem:
        pltpu.sync_copy(acc.at[pl.ds(zbase + nfull * kk, rem)],
                        out_hbm.at[pl.ds(obase + nfull * kk, rem)])


def _make_edge_kernel(A, D, n_chunks):
    rpt = A // NS

    @functools.partial(
        pl.kernel,
        out_type=jax.ShapeDtypeStruct((NC * A, D), jnp.float32),
        mesh=_mesh(),
        compiler_params=pltpu.CompilerParams(use_tc_tiling_on_sc=False),
        scratch_types=[
            pltpu.VMEM((K,), jnp.int32),
            pltpu.VMEM((K,), jnp.int32),
            pltpu.VMEM((K, D), jnp.float32),
            pltpu.VMEM_SHARED((A, D), jnp.float32),
            pltpu.SemaphoreType.DMA,
        ],
    )
    def k(proj_hbm, src_hbm, dst_hbm, out_hbm, sidx, didx, rows, acc, sem):
        c = lax.axis_index("c")
        s = lax.axis_index("s")
        wid = s * NC + c
        zbase = s * rpt
        _zero_acc(rows, acc, zbase, rpt, K)
        plsc.subcore_barrier()
        ebase = wid * (n_chunks * K)

        @pl.loop(0, n_chunks)
        def _(i):
            off = ebase + i * K
            pltpu.sync_copy(src_hbm.at[pl.ds(off, K)], sidx)
            pltpu.sync_copy(dst_hbm.at[pl.ds(off, K)], didx)
            pltpu.async_copy(proj_hbm.at[sidx], rows, sem).wait()
            pltpu.sync_copy(rows, acc.at[didx], add=True)

        plsc.subcore_barrier()
        _write_out(acc, out_hbm, c * A + zbase, zbase, rpt, K)

    return k


def _make_deg_kernel(A, D, n_chunks):
    rpt = A // NS

    @functools.partial(
        pl.kernel,
        out_type=jax.ShapeDtypeStruct((NC * A, D), jnp.float32),
        mesh=_mesh(),
        compiler_params=pltpu.CompilerParams(use_tc_tiling_on_sc=False),
        scratch_types=[
            pltpu.VMEM((K,), jnp.int32),
            pltpu.VMEM((K, D), jnp.float32),
            pltpu.VMEM_SHARED((A, D), jnp.float32),
        ],
    )
    def k(dst_hbm, out_hbm, didx, rows, acc):
        c = lax.axis_index("c")
        s = lax.axis_index("s")
        wid = s * NC + c
        zbase = s * rpt
        _zero_acc(rows, acc, zbase, rpt, K)
        plsc.subcore_barrier()

        @pl.loop(0, K)
        def _(i):
            rows[i, :] = jnp.ones((16,), jnp.float32)

        ebase = wid * (n_chunks * K)

        @pl.loop(0, n_chunks)
        def _(i):
            off = ebase + i * K
            pltpu.sync_copy(dst_hbm.at[pl.ds(off, K)], didx)
            pltpu.sync_copy(rows, acc.at[didx], add=True)

        plsc.subcore_barrier()
        _write_out(acc, out_hbm, c * A + zbase, zbase, rpt, K)

    return k


def _mm0_body(x_ref, wm_ref, wf_ref, b_ref, proj_ref, film_ref):
    h = x_ref[...]
    proj_ref[...] = jnp.dot(h, wm_ref[...], preferred_element_type=jnp.float32)
    film_ref[...] = (jnp.dot(h, wf_ref[...], preferred_element_type=jnp.float32)
                     + b_ref[0:1, :])


def _combine(s0_ref, s1_ref, f_ref, d0_ref, d1_ref, D):
    Ssum = s0_ref[0] + s1_ref[0]
    deg = d0_ref[0] + d1_ref[0]
    gamma = f_ref[:, :D]
    beta = f_ref[:, D:]
    return jnp.maximum(gamma * Ssum + deg * beta, 0.0)


def _make_mm0(N, D):
    grid = N // BR
    return pl.pallas_call(
        _mm0_body,
        grid=(grid,),
        in_specs=[
            pl.BlockSpec((BR, D), lambda i: (i, 0)),
            pl.BlockSpec((D, D), lambda i: (0, 0)),
            pl.BlockSpec((D, 2 * D), lambda i: (0, 0)),
            pl.BlockSpec((8, 2 * D), lambda i: (0, 0)),
        ],
        out_specs=[
            pl.BlockSpec((BR, D), lambda i: (i, 0)),
            pl.BlockSpec((BR, 2 * D), lambda i: (i, 0)),
        ],
        out_shape=[
            jax.ShapeDtypeStruct((N, D), jnp.float32),
            jax.ShapeDtypeStruct((N, 2 * D), jnp.float32),
        ],
    )


def _make_mmc(N, A, D):
    grid = N // BR
    nblk_a = A // BR

    def body(s_ref, s1_ref, f_ref, d0_ref, d1_ref, wm_ref, wf_ref, b_ref,
             proj_ref, film_ref):
        h = _combine(s_ref, s1_ref, f_ref, d0_ref, d1_ref, D)
        proj_ref[...] = jnp.dot(h, wm_ref[...],
                                preferred_element_type=jnp.float32)
        film_ref[...] = (jnp.dot(h, wf_ref[...],
                                 preferred_element_type=jnp.float32)
                         + b_ref[0:1, :])

    sp = pl.BlockSpec((1, BR, D), lambda i: (0, i, 0))
    return pl.pallas_call(
        body,
        grid=(grid,),
        in_specs=[
            sp,
            pl.BlockSpec((1, BR, D), lambda i: (1, i, 0)),
            pl.BlockSpec((BR, 2 * D), lambda i: (i, 0)),
            sp,
            pl.BlockSpec((1, BR, D), lambda i: (1, i, 0)),
            pl.BlockSpec((D, D), lambda i: (0, 0)),
            pl.BlockSpec((D, 2 * D), lambda i: (0, 0)),
            pl.BlockSpec((8, 2 * D), lambda i: (0, 0)),
        ],
        out_specs=[
            pl.BlockSpec((BR, D), lambda i: (i, 0)),
            pl.BlockSpec((BR, 2 * D), lambda i: (i, 0)),
        ],
        out_shape=[
            jax.ShapeDtypeStruct((N, D), jnp.float32),
            jax.ShapeDtypeStruct((N, 2 * D), jnp.float32),
        ],
    )


def _make_mmf(N, A, D):
    grid = N // BR

    def body(s_ref, s1_ref, f_ref, d0_ref, d1_ref, h_ref):
        h_ref[...] = _combine(s_ref, s1_ref, f_ref, d0_ref, d1_ref, D)

    return pl.pallas_call(
        body,
        grid=(grid,),
        in_specs=[
            pl.BlockSpec((1, BR, D), lambda i: (0, i, 0)),
            pl.BlockSpec((1, BR, D), lambda i: (1, i, 0)),
            pl.BlockSpec((BR, 2 * D), lambda i: (i, 0)),
            pl.BlockSpec((1, BR, D), lambda i: (0, i, 0)),
            pl.BlockSpec((1, BR, D), lambda i: (1, i, 0)),
        ],
        out_specs=[pl.BlockSpec((BR, D), lambda i: (i, 0))],
        out_shape=[jax.ShapeDtypeStruct((N, D), jnp.float32)],
    )


def kernel(x, edge_index, W_msg, W_film, b_film):
    N, D = x.shape
    E = edge_index.shape[1]
    L = W_msg.shape[0]
    assert D == 16

    per = -(-E // NW)
    n_chunks = -(-per // K)
    per_pad = n_chunks * K
    E_pad = NW * per_pad

    src = edge_index[0]
    dst = edge_index[1]
    if E_pad == E:
        A = -(-N // NS) * NS
    else:
        A = -(-(N + 1) // NS) * NS
        pos = jnp.arange(E_pad, dtype=jnp.int32)
        w = pos // per_pad
        p = pos % per_pad
        g = w * per + p
        valid = (p < per) & (g < E)
        gc = jnp.minimum(g, E - 1)
        src = jnp.where(valid, src[gc], 0)
        dst = jnp.where(valid, dst[gc], N)

    edge_k = _make_edge_kernel(A, D, n_chunks)
    deg_k = _make_deg_kernel(A, D, n_chunks)
    mm0 = _make_mm0(N, D)
    mmc = _make_mmc(N, A, D)
    mmf = _make_mmf(N, A, D)

    b2 = jnp.broadcast_to(b_film[:, None, :], (L, 8, 2 * D))

    degp = deg_k(dst).reshape(NC, A, D)
    proj, film = mm0(x, W_msg[0], W_film[0], b2[0])
    for l in range(L):
        Sp = edge_k(proj, src, dst).reshape(NC, A, D)
        if l < L - 1:
            proj, film = mmc(Sp, Sp, film, degp, degp,
                             W_msg[l + 1], W_film[l + 1], b2[l + 1])
        else:
            (h,) = mmf(Sp, Sp, film, degp, degp)
    return h


# speedup vs baseline: 1.4353x; 1.2527x over previous
import functools

import jax
import jax.numpy as jnp
from jax import lax
from jax.experimental import pallas as pl
from jax.experimental.pallas import tpu as pltpu
from jax.experimental.pallas import tpu_sc as plsc

NC = 2
NS = 16
NW = NC * NS
K = 800
BR = 2000


def _mesh():
    return plsc.VectorSubcoreMesh(
        core_axis_name="c", subcore_axis_name="s", num_cores=NC, num_subcores=NS)


def _zero_acc(rows, acc, zbase, rpt, kk):
    @pl.loop(0, kk)
    def _(i):
        rows[i, :] = jnp.zeros((16,), jnp.float32)
    nfull, rem = rpt // kk, rpt % kk
    for j in range(nfull):
        pltpu.sync_copy(rows, acc.at[pl.ds(zbase + j * kk, kk)])
    if rem:
        pltpu.sync_copy(rows.at[pl.ds(0, rem)],
                        acc.at[pl.ds(zbase + nfull * kk, rem)])


def _write_out(acc, out_hbm, obase, zbase, rpt, kk):
    nfull, rem = rpt // kk, rpt % kk
    for j in range(nfull):
        pltpu.sync_copy(acc.at[pl.ds(zbase + j * kk, kk)],
                        out_hbm.at[pl.ds(obase + j * kk, kk)])
    if rem:
        pltpu.sync_copy(acc.at[pl.ds(zbase + nfull * kk, rem)],
                        out_hbm.at[pl.ds(obase + nfull * kk, rem)])


def _make_edge_kernel(A, D, n_chunks):
    rpt = A // NS

    @functools.partial(
        pl.kernel,
        out_type=jax.ShapeDtypeStruct((NC * A, D), jnp.float32),
        mesh=_mesh(),
        compiler_params=pltpu.CompilerParams(use_tc_tiling_on_sc=False),
        scratch_types=[
            pltpu.VMEM((K,), jnp.int32),
            pltpu.VMEM((K,), jnp.int32),
            pltpu.VMEM((K,), jnp.int32),
            pltpu.VMEM((K,), jnp.int32),
            pltpu.VMEM((K, D), jnp.float32),
            pltpu.VMEM((K, D), jnp.float32),
            pltpu.VMEM_SHARED((A, D), jnp.float32),
            pltpu.SemaphoreType.DMA,
            pltpu.SemaphoreType.DMA,
        ],
    )
    def k(proj_hbm, src_hbm, dst_hbm, out_hbm,
          sidx0, sidx1, didx0, didx1, rows0, rows1, acc, sem0, sem1):
        c = lax.axis_index("c")
        s = lax.axis_index("s")
        wid = s * NC + c
        zbase = s * rpt
        _zero_acc(rows0, acc, zbase, rpt, K)
        plsc.subcore_barrier()
        ebase = wid * (n_chunks * K)
        sidx = (sidx0, sidx1)
        didx = (didx0, didx1)
        rows = (rows0, rows1)
        sem = (sem0, sem1)

        def fetch(i, b):
            off = ebase + i * K
            pltpu.sync_copy(src_hbm.at[pl.ds(off, K)], sidx[b])
            pltpu.sync_copy(dst_hbm.at[pl.ds(off, K)], didx[b])
            pltpu.async_copy(proj_hbm.at[sidx[b]], rows[b], sem[b])

        def drain_scatter(b):
            pltpu.make_async_copy(proj_hbm.at[sidx[b]], rows[b], sem[b]).wait()
            pltpu.sync_copy(rows[b], acc.at[didx[b]], add=True)

        fetch(0, 0)
        m = (n_chunks - 1) // 2

        @pl.loop(0, m)
        def _(t):
            for b in (0, 1):
                fetch(2 * t + b + 1, 1 - b)
                drain_scatter(b)

        for i in range(2 * m, n_chunks):
            b = i % 2
            if i + 1 < n_chunks:
                fetch(i + 1, 1 - b)
            drain_scatter(b)

        plsc.subcore_barrier()
        _write_out(acc, out_hbm, c * A + zbase, zbase, rpt, K)

    return k


def _make_deg_kernel(A, D, n_chunks):
    rpt = A // NS

    @functools.partial(
        pl.kernel,
        out_type=jax.ShapeDtypeStruct((NC * A, D), jnp.float32),
        mesh=_mesh(),
        compiler_params=pltpu.CompilerParams(use_tc_tiling_on_sc=False),
        scratch_types=[
            pltpu.VMEM((K,), jnp.int32),
            pltpu.VMEM((K, D), jnp.float32),
            pltpu.VMEM_SHARED((A, D), jnp.float32),
        ],
    )
    def k(dst_hbm, out_hbm, didx, rows, acc):
        c = lax.axis_index("c")
        s = lax.axis_index("s")
        wid = s * NC + c
        zbase = s * rpt
        _zero_acc(rows, acc, zbase, rpt, K)
        plsc.subcore_barrier()

        @pl.loop(0, K)
        def _(i):
            rows[i, :] = jnp.ones((16,), jnp.float32)

        ebase = wid * (n_chunks * K)

        @pl.loop(0, n_chunks)
        def _(i):
            off = ebase + i * K
            pltpu.sync_copy(dst_hbm.at[pl.ds(off, K)], didx)
            pltpu.sync_copy(rows, acc.at[didx], add=True)

        plsc.subcore_barrier()
        _write_out(acc, out_hbm, c * A + zbase, zbase, rpt, K)

    return k


def _mm0_body(x_ref, wm_ref, wf_ref, b_ref, proj_ref, film_ref):
    h = x_ref[...]
    proj_ref[...] = jnp.dot(h, wm_ref[...], preferred_element_type=jnp.float32)
    film_ref[...] = (jnp.dot(h, wf_ref[...], preferred_element_type=jnp.float32)
                     + b_ref[0:1, :])


def _combine(s0_ref, s1_ref, f_ref, d0_ref, d1_ref, D):
    Ssum = s0_ref[0] + s1_ref[0]
    deg = d0_ref[0] + d1_ref[0]
    gamma = f_ref[:, :D]
    beta = f_ref[:, D:]
    return jnp.maximum(gamma * Ssum + deg * beta, 0.0)


def _make_mm0(N, D):
    grid = N // BR
    return pl.pallas_call(
        _mm0_body,
        grid=(grid,),
        in_specs=[
            pl.BlockSpec((BR, D), lambda i: (i, 0)),
            pl.BlockSpec((D, D), lambda i: (0, 0)),
            pl.BlockSpec((D, 2 * D), lambda i: (0, 0)),
            pl.BlockSpec((8, 2 * D), lambda i: (0, 0)),
        ],
        out_specs=[
            pl.BlockSpec((BR, D), lambda i: (i, 0)),
            pl.BlockSpec((BR, 2 * D), lambda i: (i, 0)),
        ],
        out_shape=[
            jax.ShapeDtypeStruct((N, D), jnp.float32),
            jax.ShapeDtypeStruct((N, 2 * D), jnp.float32),
        ],
    )


def _make_mmc(N, A, D):
    grid = N // BR
    nblk_a = A // BR

    def body(s_ref, s1_ref, f_ref, d0_ref, d1_ref, wm_ref, wf_ref, b_ref,
             proj_ref, film_ref):
        h = _combine(s_ref, s1_ref, f_ref, d0_ref, d1_ref, D)
        proj_ref[...] = jnp.dot(h, wm_ref[...],
                                preferred_element_type=jnp.float32)
        film_ref[...] = (jnp.dot(h, wf_ref[...],
                                 preferred_element_type=jnp.float32)
                         + b_ref[0:1, :])

    sp = pl.BlockSpec((1, BR, D), lambda i: (0, i, 0))
    return pl.pallas_call(
        body,
        grid=(grid,),
        in_specs=[
            sp,
            pl.BlockSpec((1, BR, D), lambda i: (1, i, 0)),
            pl.BlockSpec((BR, 2 * D), lambda i: (i, 0)),
            sp,
            pl.BlockSpec((1, BR, D), lambda i: (1, i, 0)),
            pl.BlockSpec((D, D), lambda i: (0, 0)),
            pl.BlockSpec((D, 2 * D), lambda i: (0, 0)),
            pl.BlockSpec((8, 2 * D), lambda i: (0, 0)),
        ],
        out_specs=[
            pl.BlockSpec((BR, D), lambda i: (i, 0)),
            pl.BlockSpec((BR, 2 * D), lambda i: (i, 0)),
        ],
        out_shape=[
            jax.ShapeDtypeStruct((N, D), jnp.float32),
            jax.ShapeDtypeStruct((N, 2 * D), jnp.float32),
        ],
    )


def _make_mmf(N, A, D):
    grid = N // BR

    def body(s_ref, s1_ref, f_ref, d0_ref, d1_ref, h_ref):
        h_ref[...] = _combine(s_ref, s1_ref, f_ref, d0_ref, d1_ref, D)

    return pl.pallas_call(
        body,
        grid=(grid,),
        in_specs=[
            pl.BlockSpec((1, BR, D), lambda i: (0, i, 0)),
            pl.BlockSpec((1, BR, D), lambda i: (1, i, 0)),
            pl.BlockSpec((BR, 2 * D), lambda i: (i, 0)),
            pl.BlockSpec((1, BR, D), lambda i: (0, i, 0)),
            pl.BlockSpec((1, BR, D), lambda i: (1, i, 0)),
        ],
        out_specs=[pl.BlockSpec((BR, D), lambda i: (i, 0))],
        out_shape=[jax.ShapeDtypeStruct((N, D), jnp.float32)],
    )


def kernel(x, edge_index, W_msg, W_film, b_film):
    N, D = x.shape
    E = edge_index.shape[1]
    L = W_msg.shape[0]
    assert D == 16

    per = -(-E // NW)
    n_chunks = -(-per // K)
    per_pad = n_chunks * K
    E_pad = NW * per_pad

    src = edge_index[0]
    dst = edge_index[1]
    if E_pad == E:
        A = -(-N // NS) * NS
    else:
        A = -(-(N + 1) // NS) * NS
        pos = jnp.arange(E_pad, dtype=jnp.int32)
        w = pos // per_pad
        p = pos % per_pad
        g = w * per + p
        valid = (p < per) & (g < E)
        gc = jnp.minimum(g, E - 1)
        src = jnp.where(valid, src[gc], 0)
        dst = jnp.where(valid, dst[gc], N)

    edge_k = _make_edge_kernel(A, D, n_chunks)
    deg_k = _make_deg_kernel(A, D, n_chunks)
    mm0 = _make_mm0(N, D)
    mmc = _make_mmc(N, A, D)
    mmf = _make_mmf(N, A, D)

    b2 = jnp.broadcast_to(b_film[:, None, :], (L, 8, 2 * D))

    degp = deg_k(dst).reshape(NC, A, D)
    proj, film = mm0(x, W_msg[0], W_film[0], b2[0])
    for l in range(L):
        Sp = edge_k(proj, src, dst).reshape(NC, A, D)
        if l < L - 1:
            proj, film = mmc(Sp, Sp, film, degp, degp,
                             W_msg[l + 1], W_film[l + 1], b2[l + 1])
        else:
            (h,) = mmf(Sp, Sp, film, degp, degp)
    return h


# speedup vs baseline: 1.6712x; 1.1644x over previous
import functools

import jax
import jax.numpy as jnp
from jax import lax
from jax.experimental import pallas as pl
from jax.experimental.pallas import tpu as pltpu
from jax.experimental.pallas import tpu_sc as plsc

NC = 2
NS = 16
NW = NC * NS
K = 800


def _mesh():
    return plsc.VectorSubcoreMesh(
        core_axis_name="c", subcore_axis_name="s", num_cores=NC, num_subcores=NS)


def _zero_acc(rows, acc, zbase, rpt, kk):
    @pl.loop(0, kk)
    def _(i):
        rows[i, :] = jnp.zeros((16,), jnp.float32)
    nfull, rem = rpt // kk, rpt % kk
    for j in range(nfull):
        pltpu.sync_copy(rows, acc.at[pl.ds(zbase + j * kk, kk)])
    if rem:
        pltpu.sync_copy(rows.at[pl.ds(0, rem)],
                        acc.at[pl.ds(zbase + nfull * kk, rem)])


def _write_out(acc, out_hbm, obase, zbase, rpt, kk):
    nfull, rem = rpt // kk, rpt % kk
    for j in range(nfull):
        pltpu.sync_copy(acc.at[pl.ds(zbase + j * kk, kk)],
                        out_hbm.at[pl.ds(obase + j * kk, kk)])
    if rem:
        pltpu.sync_copy(acc.at[pl.ds(zbase + nfull * kk, rem)],
                        out_hbm.at[pl.ds(obase + nfull * kk, rem)])


def _make_edge_kernel(A, D, n_chunks):
    rpt = A // NS

    @functools.partial(
        pl.kernel,
        out_type=jax.ShapeDtypeStruct((NC * A, D), jnp.float32),
        mesh=_mesh(),
        compiler_params=pltpu.CompilerParams(use_tc_tiling_on_sc=False),
        scratch_types=[
            pltpu.VMEM((K,), jnp.int32),
            pltpu.VMEM((K,), jnp.int32),
            pltpu.VMEM((K,), jnp.int32),
            pltpu.VMEM((K,), jnp.int32),
            pltpu.VMEM((K, D), jnp.float32),
            pltpu.VMEM((K, D), jnp.float32),
            pltpu.VMEM_SHARED((A, D), jnp.float32),
            pltpu.SemaphoreType.DMA,
            pltpu.SemaphoreType.DMA,
        ],
    )
    def k(proj_hbm, src_hbm, dst_hbm, out_hbm,
          sidx0, sidx1, didx0, didx1, rows0, rows1, acc, sem0, sem1):
        c = lax.axis_index("c")
        s = lax.axis_index("s")
        wid = s * NC + c
        zbase = s * rpt
        _zero_acc(rows0, acc, zbase, rpt, K)
        plsc.subcore_barrier()
        ebase = wid * (n_chunks * K)
        sidx = (sidx0, sidx1)
        didx = (didx0, didx1)
        rows = (rows0, rows1)
        sem = (sem0, sem1)

        def fetch(i, b):
            off = ebase + i * K
            pltpu.sync_copy(src_hbm.at[pl.ds(off, K)], sidx[b])
            pltpu.sync_copy(dst_hbm.at[pl.ds(off, K)], didx[b])
            pltpu.async_copy(proj_hbm.at[sidx[b]], rows[b], sem[b])

        def drain_scatter(b):
            pltpu.make_async_copy(proj_hbm.at[sidx[b]], rows[b], sem[b]).wait()
            pltpu.sync_copy(rows[b], acc.at[didx[b]], add=True)

        fetch(0, 0)
        m = (n_chunks - 1) // 2

        @pl.loop(0, m)
        def _(t):
            for b in (0, 1):
                fetch(2 * t + b + 1, 1 - b)
                drain_scatter(b)

        for i in range(2 * m, n_chunks):
            b = i % 2
            if i + 1 < n_chunks:
                fetch(i + 1, 1 - b)
            drain_scatter(b)

        plsc.subcore_barrier()
        _write_out(acc, out_hbm, c * A + zbase, zbase, rpt, K)

    return k


def _make_deg_kernel(A, D, n_chunks):
    rpt = A // NS

    @functools.partial(
        pl.kernel,
        out_type=jax.ShapeDtypeStruct((NC * A, D), jnp.float32),
        mesh=_mesh(),
        compiler_params=pltpu.CompilerParams(use_tc_tiling_on_sc=False),
        scratch_types=[
            pltpu.VMEM((K,), jnp.int32),
            pltpu.VMEM((K, D), jnp.float32),
            pltpu.VMEM_SHARED((A, D), jnp.float32),
        ],
    )
    def k(dst_hbm, out_hbm, didx, rows, acc):
        c = lax.axis_index("c")
        s = lax.axis_index("s")
        wid = s * NC + c
        zbase = s * rpt
        _zero_acc(rows, acc, zbase, rpt, K)
        plsc.subcore_barrier()

        @pl.loop(0, K)
        def _(i):
            rows[i, :] = jnp.ones((16,), jnp.float32)

        ebase = wid * (n_chunks * K)

        @pl.loop(0, n_chunks)
        def _(i):
            off = ebase + i * K
            pltpu.sync_copy(dst_hbm.at[pl.ds(off, K)], didx)
            pltpu.sync_copy(rows, acc.at[didx], add=True)

        plsc.subcore_barrier()
        _write_out(acc, out_hbm, c * A + zbase, zbase, rpt, K)

    return k


def _make_node_kernel(N, A, D, combine, matmul):
    npt = N // NW
    CH = 625 if npt % 625 == 0 else npt
    n_ch = npt // CH
    assert CH % 5 == 0 and npt == CH * n_ch

    out_type = [jax.ShapeDtypeStruct((N, D), jnp.float32)]
    if matmul:
        out_type.append(jax.ShapeDtypeStruct((N, 2 * D), jnp.float32))

    scratch = []
    if combine:
        scratch += [pltpu.VMEM((CH, D), jnp.float32),
                    pltpu.VMEM((CH, D), jnp.float32),
                    pltpu.VMEM((CH, D), jnp.float32),
                    pltpu.VMEM((CH, D), jnp.float32),
                    pltpu.VMEM((CH, 2 * D), jnp.float32)]
    else:
        scratch += [pltpu.VMEM((CH, D), jnp.float32)]
    scratch += [pltpu.VMEM((CH, D), jnp.float32)]
    if matmul:
        scratch += [pltpu.VMEM((D, D), jnp.float32),
                    pltpu.VMEM((D, 2 * D), jnp.float32),
                    pltpu.VMEM((2 * D,), jnp.float32),
                    pltpu.VMEM((CH, 2 * D), jnp.float32),
                    pltpu.VMEM((80,), jnp.float32)]

    @functools.partial(
        pl.kernel,
        out_type=tuple(out_type) if matmul else out_type[0],
        mesh=_mesh(),
        compiler_params=pltpu.CompilerParams(use_tc_tiling_on_sc=False,
                                             needs_layout_passes=False),
        scratch_types=scratch,
    )
    def k(*refs):
        it = iter(refs)
        if combine:
            sp_hbm = next(it)
            deg_hbm = next(it)
            film_hbm = next(it)
        else:
            x_hbm = next(it)
        if matmul:
            wm_hbm = next(it)
            wf_hbm = next(it)
            b_hbm = next(it)
        h_out_hbm = next(it)
        if matmul:
            film_out_hbm = next(it)
        if combine:
            s0b = next(it); s1b = next(it); d0b = next(it); d1b = next(it)
            fb = next(it)
        else:
            xb = next(it)
        hb = next(it)
        if matmul:
            wmb = next(it); wfb = next(it); bb = next(it)
            fob = next(it); h5 = next(it)

        c = lax.axis_index("c")
        s = lax.axis_index("s")
        wid = s * NC + c
        nbase = wid * npt
        if matmul:
            pltpu.sync_copy(wm_hbm, wmb)
            pltpu.sync_copy(wf_hbm, wfb)
            pltpu.sync_copy(b_hbm, bb)

        for ch in range(n_ch):
            gbase = nbase + ch * CH
            if combine:
                pltpu.sync_copy(sp_hbm.at[pl.ds(gbase, CH)], s0b)
                pltpu.sync_copy(sp_hbm.at[pl.ds(A + gbase, CH)], s1b)
                pltpu.sync_copy(deg_hbm.at[pl.ds(gbase, CH)], d0b)
                pltpu.sync_copy(deg_hbm.at[pl.ds(A + gbase, CH)], d1b)
                pltpu.sync_copy(film_hbm.at[pl.ds(gbase, CH)], fb)
            else:
                pltpu.sync_copy(x_hbm.at[pl.ds(gbase, CH)], xb)

            @pl.loop(0, CH // 5)
            def _(t):
                r0 = t * 5
                for ri in range(5):
                    r = r0 + ri
                    if combine:
                        gam = fb[r, 0:D]
                        bet = fb[r, D:2 * D]
                        h = jnp.maximum(
                            gam * (s0b[r, :] + s1b[r, :])
                            + (d0b[r, :] + d1b[r, :]) * bet, 0.0)
                    else:
                        h = xb[r, :]
                    if matmul:
                        h5[pl.ds(ri * 16, 16)] = h
                    else:
                        hb[r, :] = h
                if matmul:
                    pj = [jnp.zeros((16,), jnp.float32) for _ in range(5)]
                    fg = [bb[0:D]] * 5
                    fbeta = [bb[D:2 * D]] * 5
                    for j in range(D):
                        wmr = wmb[j, :]
                        wgr = wfb[j, 0:D]
                        wbr = wfb[j, D:2 * D]
                        for ri in range(5):
                            hj = plsc.load_gather(
                                h5, [jnp.full((16,), ri * 16 + j, jnp.int32)])
                            pj[ri] = pj[ri] + hj * wmr
                            fg[ri] = fg[ri] + hj * wgr
                            fbeta[ri] = fbeta[ri] + hj * wbr
                    for ri in range(5):
                        r = r0 + ri
                        hb[r, :] = pj[ri]
                        fob[r, 0:D] = fg[ri]
                        fob[r, D:2 * D] = fbeta[ri]

            if matmul:
                pltpu.sync_copy(hb, h_out_hbm.at[pl.ds(gbase, CH)])
                pltpu.sync_copy(fob, film_out_hbm.at[pl.ds(gbase, CH)])
            else:
                pltpu.sync_copy(hb, h_out_hbm.at[pl.ds(gbase, CH)])

    return k


def kernel(x, edge_index, W_msg, W_film, b_film):
    N, D = x.shape
    E = edge_index.shape[1]
    L = W_msg.shape[0]
    assert D == 16

    per = -(-E // NW)
    n_chunks = -(-per // K)
    per_pad = n_chunks * K
    E_pad = NW * per_pad

    src = edge_index[0]
    dst = edge_index[1]
    if E_pad == E:
        A = -(-N // NS) * NS
    else:
        A = -(-(N + 1) // NS) * NS
        pos = jnp.arange(E_pad, dtype=jnp.int32)
        w = pos // per_pad
        p = pos % per_pad
        g = w * per + p
        valid = (p < per) & (g < E)
        gc = jnp.minimum(g, E - 1)
        src = jnp.where(valid, src[gc], 0)
        dst = jnp.where(valid, dst[gc], N)

    assert N % NW == 0
    edge_k = _make_edge_kernel(A, D, n_chunks)
    deg_k = _make_deg_kernel(A, D, n_chunks)
    mm0 = _make_node_kernel(N, A, D, combine=False, matmul=True)
    mmc = _make_node_kernel(N, A, D, combine=True, matmul=True)
    mmf = _make_node_kernel(N, A, D, combine=True, matmul=False)

    degp = deg_k(dst)
    proj, film = mm0(x, W_msg[0], W_film[0], b_film[0])
    for l in range(L):
        Sp = edge_k(proj, src, dst)
        if l < L - 1:
            proj, film = mmc(Sp, degp, film,
                             W_msg[l + 1], W_film[l + 1], b_film[l + 1])
        else:
            h = mmf(Sp, degp, film)
    return h


# speedup vs baseline: 1.7357x; 1.0386x over previous
import functools

import jax
import jax.numpy as jnp
from jax import lax
from jax.experimental import pallas as pl
from jax.experimental.pallas import tpu as pltpu
from jax.experimental.pallas import tpu_sc as plsc

NC = 2
NS = 16
NW = NC * NS
K = 800


def _mesh():
    return plsc.VectorSubcoreMesh(
        core_axis_name="c", subcore_axis_name="s", num_cores=NC, num_subcores=NS)


def _zero_acc(rows, acc, zbase, rpt, kk):
    @pl.loop(0, kk)
    def _(i):
        rows[i, :] = jnp.zeros((16,), jnp.float32)
    nfull, rem = rpt // kk, rpt % kk
    for j in range(nfull):
        pltpu.sync_copy(rows, acc.at[pl.ds(zbase + j * kk, kk)])
    if rem:
        pltpu.sync_copy(rows.at[pl.ds(0, rem)],
                        acc.at[pl.ds(zbase + nfull * kk, rem)])


def _write_out(acc, out_hbm, obase, zbase, rpt, kk):
    nfull, rem = rpt // kk, rpt % kk
    for j in range(nfull):
        pltpu.sync_copy(acc.at[pl.ds(zbase + j * kk, kk)],
                        out_hbm.at[pl.ds(obase + j * kk, kk)])
    if rem:
        pltpu.sync_copy(acc.at[pl.ds(zbase + nfull * kk, rem)],
                        out_hbm.at[pl.ds(obase + nfull * kk, rem)])


def _make_edge_kernel(A, D, n_chunks):
    rpt = A // NS

    @functools.partial(
        pl.kernel,
        out_type=jax.ShapeDtypeStruct((NC * A, D), jnp.float32),
        mesh=_mesh(),
        compiler_params=pltpu.CompilerParams(use_tc_tiling_on_sc=False),
        scratch_types=[
            pltpu.VMEM((K,), jnp.int32),
            pltpu.VMEM((K,), jnp.int32),
            pltpu.VMEM((K,), jnp.int32),
            pltpu.VMEM((K,), jnp.int32),
            pltpu.VMEM((K, D), jnp.float32),
            pltpu.VMEM((K, D), jnp.float32),
            pltpu.VMEM_SHARED((A, D), jnp.float32),
            pltpu.SemaphoreType.DMA,
            pltpu.SemaphoreType.DMA,
            pltpu.SemaphoreType.DMA,
            pltpu.SemaphoreType.DMA,
        ],
    )
    def k(proj_hbm, src_hbm, dst_hbm, out_hbm,
          sidx0, sidx1, didx0, didx1, rows0, rows1, acc,
          sem0, sem1, ssem0, ssem1):
        c = lax.axis_index("c")
        s = lax.axis_index("s")
        wid = s * NC + c
        zbase = s * rpt
        _zero_acc(rows0, acc, zbase, rpt, K)
        plsc.subcore_barrier()
        ebase = wid * (n_chunks * K)
        sidx = (sidx0, sidx1)
        didx = (didx0, didx1)
        rows = (rows0, rows1)
        sem = (sem0, sem1)
        ssem = (ssem0, ssem1)

        def fetch(i, b):
            off = ebase + i * K
            pltpu.sync_copy(src_hbm.at[pl.ds(off, K)], sidx[b])
            pltpu.sync_copy(dst_hbm.at[pl.ds(off, K)], didx[b])
            pltpu.async_copy(proj_hbm.at[sidx[b]], rows[b], sem[b])

        def scatter(b):
            pltpu.make_async_copy(proj_hbm.at[sidx[b]], rows[b], sem[b]).wait()
            pltpu.async_copy(rows[b], acc.at[didx[b]], ssem[b], add=True)

        def wait_scatter(b):
            pltpu.make_async_copy(rows[b], acc.at[didx[b]], ssem[b]).wait()

        assert n_chunks >= 4
        fetch(0, 0)
        fetch(1, 1)
        scatter(0)
        m2 = (n_chunks - 2) // 2

        @pl.loop(1, m2 + 1)
        def _(t):
            for b in (0, 1):
                i = 2 * t + b
                wait_scatter(b)
                fetch(i, b)
                scatter(1 - b)

        for i in range(2 * m2 + 2, n_chunks):
            b = i % 2
            wait_scatter(b)
            fetch(i, b)
            scatter(1 - b)
        bl = (n_chunks - 1) % 2
        scatter(bl)
        wait_scatter(1 - bl)
        wait_scatter(bl)

        plsc.subcore_barrier()
        _write_out(acc, out_hbm, c * A + zbase, zbase, rpt, K)

    return k


def _make_deg_kernel(A, D, n_chunks):
    rpt = A // NS

    @functools.partial(
        pl.kernel,
        out_type=jax.ShapeDtypeStruct((NC * A, D), jnp.float32),
        mesh=_mesh(),
        compiler_params=pltpu.CompilerParams(use_tc_tiling_on_sc=False),
        scratch_types=[
            pltpu.VMEM((K,), jnp.int32),
            pltpu.VMEM((K,), jnp.int32),
            pltpu.VMEM((K, D), jnp.float32),
            pltpu.VMEM_SHARED((A, D), jnp.float32),
            pltpu.SemaphoreType.DMA,
            pltpu.SemaphoreType.DMA,
        ],
    )
    def k(dst_hbm, out_hbm, didx0, didx1, rows, acc, ssem0, ssem1):
        c = lax.axis_index("c")
        s = lax.axis_index("s")
        wid = s * NC + c
        zbase = s * rpt
        _zero_acc(rows, acc, zbase, rpt, K)
        plsc.subcore_barrier()

        @pl.loop(0, K)
        def _(i):
            rows[i, :] = jnp.ones((16,), jnp.float32)

        ebase = wid * (n_chunks * K)
        didx = (didx0, didx1)
        ssem = (ssem0, ssem1)

        def fetch(i, b):
            pltpu.sync_copy(dst_hbm.at[pl.ds(ebase + i * K, K)], didx[b])

        def scatter(b):
            pltpu.async_copy(rows, acc.at[didx[b]], ssem[b], add=True)

        def wait_scatter(b):
            pltpu.make_async_copy(rows, acc.at[didx[b]], ssem[b]).wait()

        assert n_chunks >= 4
        fetch(0, 0)
        fetch(1, 1)
        scatter(0)
        m2 = (n_chunks - 2) // 2

        @pl.loop(1, m2 + 1)
        def _(t):
            for b in (0, 1):
                wait_scatter(b)
                fetch(2 * t + b, b)
                scatter(1 - b)

        for i in range(2 * m2 + 2, n_chunks):
            b = i % 2
            wait_scatter(b)
            fetch(i, b)
            scatter(1 - b)
        bl = (n_chunks - 1) % 2
        scatter(bl)
        wait_scatter(1 - bl)
        wait_scatter(bl)

        plsc.subcore_barrier()
        _write_out(acc, out_hbm, c * A + zbase, zbase, rpt, K)

    return k


def _make_node_kernel(N, A, D, combine, matmul):
    npt = N // NW
    CH = 625 if npt % 625 == 0 else npt
    n_ch = npt // CH
    assert CH % 5 == 0 and npt == CH * n_ch

    out_type = [jax.ShapeDtypeStruct((N, D), jnp.float32)]
    if matmul:
        out_type.append(jax.ShapeDtypeStruct((N, 2 * D), jnp.float32))

    scratch = []
    if combine:
        scratch += [pltpu.VMEM((CH, D), jnp.float32),
                    pltpu.VMEM((CH, D), jnp.float32),
                    pltpu.VMEM((CH, D), jnp.float32),
                    pltpu.VMEM((CH, D), jnp.float32),
                    pltpu.VMEM((CH, 2 * D), jnp.float32)]
    else:
        scratch += [pltpu.VMEM((CH, D), jnp.float32)]
    scratch += [pltpu.VMEM((CH, D), jnp.float32)]
    if matmul:
        scratch += [pltpu.VMEM((D, D), jnp.float32),
                    pltpu.VMEM((D, 2 * D), jnp.float32),
                    pltpu.VMEM((2 * D,), jnp.float32),
                    pltpu.VMEM((CH, 2 * D), jnp.float32),
                    pltpu.VMEM((80,), jnp.float32)]
    scratch += [pltpu.SemaphoreType.DMA, pltpu.SemaphoreType.DMA]

    @functools.partial(
        pl.kernel,
        out_type=tuple(out_type) if matmul else out_type[0],
        mesh=_mesh(),
        compiler_params=pltpu.CompilerParams(use_tc_tiling_on_sc=False,
                                             needs_layout_passes=False),
        scratch_types=scratch,
    )
    def k(*refs):
        it = iter(refs)
        if combine:
            sp_hbm = next(it)
            deg_hbm = next(it)
            film_hbm = next(it)
        else:
            x_hbm = next(it)
        if matmul:
            wm_hbm = next(it)
            wf_hbm = next(it)
            b_hbm = next(it)
        h_out_hbm = next(it)
        if matmul:
            film_out_hbm = next(it)
        if combine:
            s0b = next(it); s1b = next(it); d0b = next(it); d1b = next(it)
            fb = next(it)
        else:
            xb = next(it)
        hb = next(it)
        if matmul:
            wmb = next(it); wfb = next(it); bb = next(it)
            fob = next(it); h5 = next(it)
        semi = next(it)
        semo = next(it)

        c = lax.axis_index("c")
        s = lax.axis_index("s")
        wid = s * NC + c
        nbase = wid * npt
        if matmul:
            pltpu.sync_copy(wm_hbm, wmb)
            pltpu.sync_copy(wf_hbm, wfb)
            pltpu.sync_copy(b_hbm, bb)

        out_prev = []
        for ch in range(n_ch):
            gbase = nbase + ch * CH
            if combine:
                ins = [(sp_hbm.at[pl.ds(gbase, CH)], s0b),
                       (sp_hbm.at[pl.ds(A + gbase, CH)], s1b),
                       (deg_hbm.at[pl.ds(gbase, CH)], d0b),
                       (deg_hbm.at[pl.ds(A + gbase, CH)], d1b),
                       (film_hbm.at[pl.ds(gbase, CH)], fb)]
            else:
                ins = [(x_hbm.at[pl.ds(gbase, CH)], xb)]
            for sref, dref in ins:
                pltpu.async_copy(sref, dref, semi)
            for dsc in out_prev:
                dsc.wait()
            for sref, dref in ins:
                pltpu.make_async_copy(sref, dref, semi).wait()

            @pl.loop(0, CH // 5)
            def _(t):
                r0 = t * 5
                for ri in range(5):
                    r = r0 + ri
                    if combine:
                        gam = fb[r, 0:D]
                        bet = fb[r, D:2 * D]
                        h = jnp.maximum(
                            gam * (s0b[r, :] + s1b[r, :])
                            + (d0b[r, :] + d1b[r, :]) * bet, 0.0)
                    else:
                        h = xb[r, :]
                    if matmul:
                        h5[pl.ds(ri * 16, 16)] = h
                    else:
                        hb[r, :] = h
                if matmul:
                    pj = [jnp.zeros((16,), jnp.float32) for _ in range(5)]
                    fg = [bb[0:D]] * 5
                    fbeta = [bb[D:2 * D]] * 5
                    for j in range(D):
                        wmr = wmb[j, :]
                        wgr = wfb[j, 0:D]
                        wbr = wfb[j, D:2 * D]
                        for ri in range(5):
                            hj = plsc.load_gather(
                                h5, [jnp.full((16,), ri * 16 + j, jnp.int32)])
                            pj[ri] = pj[ri] + hj * wmr
                            fg[ri] = fg[ri] + hj * wgr
                            fbeta[ri] = fbeta[ri] + hj * wbr
                    for ri in range(5):
                        r = r0 + ri
                        hb[r, :] = pj[ri]
                        fob[r, 0:D] = fg[ri]
                        fob[r, D:2 * D] = fbeta[ri]

            out_prev = [pltpu.async_copy(hb, h_out_hbm.at[pl.ds(gbase, CH)],
                                         semo)]
            if matmul:
                out_prev.append(
                    pltpu.async_copy(fob, film_out_hbm.at[pl.ds(gbase, CH)],
                                     semo))
        for dsc in out_prev:
            dsc.wait()

    return k


def kernel(x, edge_index, W_msg, W_film, b_film):
    N, D = x.shape
    E = edge_index.shape[1]
    L = W_msg.shape[0]
    assert D == 16

    per = -(-E // NW)
    n_chunks = -(-per // K)
    per_pad = n_chunks * K
    E_pad = NW * per_pad

    src = edge_index[0]
    dst = edge_index[1]
    if E_pad == E:
        A = -(-N // NS) * NS
    else:
        A = -(-(N + 1) // NS) * NS
        pos = jnp.arange(E_pad, dtype=jnp.int32)
        w = pos // per_pad
        p = pos % per_pad
        g = w * per + p
        valid = (p < per) & (g < E)
        gc = jnp.minimum(g, E - 1)
        src = jnp.where(valid, src[gc], 0)
        dst = jnp.where(valid, dst[gc], N)

    assert N % NW == 0
    edge_k = _make_edge_kernel(A, D, n_chunks)
    deg_k = _make_deg_kernel(A, D, n_chunks)
    mm0 = _make_node_kernel(N, A, D, combine=False, matmul=True)
    mmc = _make_node_kernel(N, A, D, combine=True, matmul=True)
    mmf = _make_node_kernel(N, A, D, combine=True, matmul=False)

    degp = deg_k(dst)
    proj, film = mm0(x, W_msg[0], W_film[0], b_film[0])
    for l in range(L):
        Sp = edge_k(proj, src, dst)
        if l < L - 1:
            proj, film = mmc(Sp, degp, film,
                             W_msg[l + 1], W_film[l + 1], b_film[l + 1])
        else:
            h = mmf(Sp, degp, film)
    return h


# speedup vs baseline: 1.8933x; 1.0908x over previous
import functools

import jax
import jax.numpy as jnp
from jax import lax
from jax.experimental import pallas as pl
from jax.experimental.pallas import tpu as pltpu
from jax.experimental.pallas import tpu_sc as plsc

NC = 2
NS = 16
NW = NC * NS
K = 800


def _mesh():
    return plsc.VectorSubcoreMesh(
        core_axis_name="c", subcore_axis_name="s", num_cores=NC, num_subcores=NS)


def _zero_acc(rows, acc, zbase, rpt, kk):
    @pl.loop(0, kk)
    def _(i):
        rows[i, :] = jnp.zeros((16,), jnp.float32)
    nfull, rem = rpt // kk, rpt % kk
    for j in range(nfull):
        pltpu.sync_copy(rows, acc.at[pl.ds(zbase + j * kk, kk)])
    if rem:
        pltpu.sync_copy(rows.at[pl.ds(0, rem)],
                        acc.at[pl.ds(zbase + nfull * kk, rem)])


def _write_out(acc, out_hbm, obase, zbase, rpt, kk):
    nfull, rem = rpt // kk, rpt % kk
    for j in range(nfull):
        pltpu.sync_copy(acc.at[pl.ds(zbase + j * kk, kk)],
                        out_hbm.at[pl.ds(obase + j * kk, kk)])
    if rem:
        pltpu.sync_copy(acc.at[pl.ds(zbase + nfull * kk, rem)],
                        out_hbm.at[pl.ds(obase + nfull * kk, rem)])


def _make_edge_kernel(A, D, n_chunks):
    rpt = A // NS

    @functools.partial(
        pl.kernel,
        out_type=jax.ShapeDtypeStruct((NC * A, D), jnp.float32),
        mesh=_mesh(),
        compiler_params=pltpu.CompilerParams(use_tc_tiling_on_sc=False),
        scratch_types=[
            pltpu.VMEM((2, 1, K), jnp.int32),
            pltpu.VMEM((2, 1, K), jnp.int32),
            pltpu.VMEM((K, D), jnp.float32),
            pltpu.VMEM((K, D), jnp.float32),
            pltpu.VMEM_SHARED((A, D), jnp.float32),
            pltpu.SemaphoreType.DMA,
            pltpu.SemaphoreType.DMA,
            pltpu.SemaphoreType.DMA,
            pltpu.SemaphoreType.DMA,
        ],
    )
    def k(proj_hbm, eidx_hbm, out_hbm,
          eb0, eb1, rows0, rows1, acc, sem0, sem1, ssem0, ssem1):
        c = lax.axis_index("c")
        s = lax.axis_index("s")
        wid = s * NC + c
        zbase = s * rpt
        _zero_acc(rows0, acc, zbase, rpt, K)
        plsc.subcore_barrier()
        cbase = wid * n_chunks
        eb = (eb0, eb1)
        rows = (rows0, rows1)
        sem = (sem0, sem1)
        ssem = (ssem0, ssem1)

        def fetch(i, b):
            pltpu.sync_copy(eidx_hbm.at[:, pl.ds(cbase + i, 1), :], eb[b])
            pltpu.async_copy(proj_hbm.at[eb[b].at[0, 0]], rows[b], sem[b])

        def scatter(b):
            pltpu.make_async_copy(proj_hbm.at[eb[b].at[0, 0]], rows[b],
                                  sem[b]).wait()
            pltpu.async_copy(rows[b], acc.at[eb[b].at[1, 0]], ssem[b],
                             add=True)

        def wait_scatter(b):
            pltpu.make_async_copy(rows[b], acc.at[eb[b].at[1, 0]],
                                  ssem[b]).wait()

        assert n_chunks >= 4
        fetch(0, 0)
        fetch(1, 1)
        scatter(0)
        m2 = (n_chunks - 2) // 2

        @pl.loop(1, m2 + 1)
        def _(t):
            for b in (0, 1):
                i = 2 * t + b
                wait_scatter(b)
                fetch(i, b)
                scatter(1 - b)

        for i in range(2 * m2 + 2, n_chunks):
            b = i % 2
            wait_scatter(b)
            fetch(i, b)
            scatter(1 - b)
        bl = (n_chunks - 1) % 2
        scatter(bl)
        wait_scatter(1 - bl)
        wait_scatter(bl)

        plsc.subcore_barrier()
        _write_out(acc, out_hbm, c * A + zbase, zbase, rpt, K)

    return k


def _make_deg_kernel(A, D, n_chunks):
    rpt = A // NS

    @functools.partial(
        pl.kernel,
        out_type=jax.ShapeDtypeStruct((NC * A, D), jnp.float32),
        mesh=_mesh(),
        compiler_params=pltpu.CompilerParams(use_tc_tiling_on_sc=False),
        scratch_types=[
            pltpu.VMEM((K,), jnp.int32),
            pltpu.VMEM((K,), jnp.int32),
            pltpu.VMEM((K, D), jnp.float32),
            pltpu.VMEM_SHARED((A, D), jnp.float32),
            pltpu.SemaphoreType.DMA,
            pltpu.SemaphoreType.DMA,
        ],
    )
    def k(dst_hbm, out_hbm, didx0, didx1, rows, acc, ssem0, ssem1):
        c = lax.axis_index("c")
        s = lax.axis_index("s")
        wid = s * NC + c
        zbase = s * rpt
        _zero_acc(rows, acc, zbase, rpt, K)
        plsc.subcore_barrier()

        @pl.loop(0, K)
        def _(i):
            rows[i, :] = jnp.ones((16,), jnp.float32)

        ebase = wid * (n_chunks * K)
        didx = (didx0, didx1)
        ssem = (ssem0, ssem1)

        def fetch(i, b):
            pltpu.sync_copy(dst_hbm.at[pl.ds(ebase + i * K, K)], didx[b])

        def scatter(b):
            pltpu.async_copy(rows, acc.at[didx[b]], ssem[b], add=True)

        def wait_scatter(b):
            pltpu.make_async_copy(rows, acc.at[didx[b]], ssem[b]).wait()

        assert n_chunks >= 4
        fetch(0, 0)
        fetch(1, 1)
        scatter(0)
        m2 = (n_chunks - 2) // 2

        @pl.loop(1, m2 + 1)
        def _(t):
            for b in (0, 1):
                wait_scatter(b)
                fetch(2 * t + b, b)
                scatter(1 - b)

        for i in range(2 * m2 + 2, n_chunks):
            b = i % 2
            wait_scatter(b)
            fetch(i, b)
            scatter(1 - b)
        bl = (n_chunks - 1) % 2
        scatter(bl)
        wait_scatter(1 - bl)
        wait_scatter(bl)

        plsc.subcore_barrier()
        _write_out(acc, out_hbm, c * A + zbase, zbase, rpt, K)

    return k


def _make_node_kernel(N, A, D, combine, matmul):
    npt = N // NW
    CH = 625 if npt % 625 == 0 else npt
    n_ch = npt // CH
    assert CH % 5 == 0 and npt == CH * n_ch

    out_type = [jax.ShapeDtypeStruct((N, D), jnp.float32)]
    if matmul:
        out_type.append(jax.ShapeDtypeStruct((N, 2 * D), jnp.float32))

    scratch = []
    if combine:
        scratch += [pltpu.VMEM((CH, D), jnp.float32),
                    pltpu.VMEM((CH, D), jnp.float32),
                    pltpu.VMEM((CH, D), jnp.float32),
                    pltpu.VMEM((CH, D), jnp.float32),
                    pltpu.VMEM((CH, 2 * D), jnp.float32)]
    else:
        scratch += [pltpu.VMEM((CH, D), jnp.float32)]
    scratch += [pltpu.VMEM((CH, D), jnp.float32)]
    if matmul:
        scratch += [pltpu.VMEM((D, D), jnp.float32),
                    pltpu.VMEM((D, 2 * D), jnp.float32),
                    pltpu.VMEM((2 * D,), jnp.float32),
                    pltpu.VMEM((CH, 2 * D), jnp.float32),
                    pltpu.VMEM((80,), jnp.float32)]
    scratch += [pltpu.SemaphoreType.DMA, pltpu.SemaphoreType.DMA]

    @functools.partial(
        pl.kernel,
        out_type=tuple(out_type) if matmul else out_type[0],
        mesh=_mesh(),
        compiler_params=pltpu.CompilerParams(use_tc_tiling_on_sc=False,
                                             needs_layout_passes=False),
        scratch_types=scratch,
    )
    def k(*refs):
        it = iter(refs)
        if combine:
            sp_hbm = next(it)
            deg_hbm = next(it)
            film_hbm = next(it)
        else:
            x_hbm = next(it)
        if matmul:
            wm_hbm = next(it)
            wf_hbm = next(it)
            b_hbm = next(it)
        h_out_hbm = next(it)
        if matmul:
            film_out_hbm = next(it)
        if combine:
            s0b = next(it); s1b = next(it); d0b = next(it); d1b = next(it)
            fb = next(it)
        else:
            xb = next(it)
        hb = next(it)
        if matmul:
            wmb = next(it); wfb = next(it); bb = next(it)
            fob = next(it); h5 = next(it)
        semi = next(it)
        semo = next(it)

        c = lax.axis_index("c")
        s = lax.axis_index("s")
        wid = s * NC + c
        nbase = wid * npt
        if matmul:
            pltpu.sync_copy(wm_hbm, wmb)
            pltpu.sync_copy(wf_hbm, wfb)
            pltpu.sync_copy(b_hbm, bb)

        out_prev = []
        for ch in range(n_ch):
            gbase = nbase + ch * CH
            if combine:
                ins = [(sp_hbm.at[pl.ds(gbase, CH)], s0b),
                       (sp_hbm.at[pl.ds(A + gbase, CH)], s1b),
                       (deg_hbm.at[pl.ds(gbase, CH)], d0b),
                       (deg_hbm.at[pl.ds(A + gbase, CH)], d1b),
                       (film_hbm.at[pl.ds(gbase, CH)], fb)]
            else:
                ins = [(x_hbm.at[pl.ds(gbase, CH)], xb)]
            for sref, dref in ins:
                pltpu.async_copy(sref, dref, semi)
            for dsc in out_prev:
                dsc.wait()
            for sref, dref in ins:
                pltpu.make_async_copy(sref, dref, semi).wait()

            @pl.loop(0, CH // 5)
            def _(t):
                r0 = t * 5
                for ri in range(5):
                    r = r0 + ri
                    if combine:
                        gam = fb[r, 0:D]
                        bet = fb[r, D:2 * D]
                        h = jnp.maximum(
                            gam * (s0b[r, :] + s1b[r, :])
                            + (d0b[r, :] + d1b[r, :]) * bet, 0.0)
                    else:
                        h = xb[r, :]
                    if matmul:
                        h5[pl.ds(ri * 16, 16)] = h
                    else:
                        hb[r, :] = h
                if matmul:
                    pj = [jnp.zeros((16,), jnp.float32) for _ in range(5)]
                    fg = [bb[0:D]] * 5
                    fbeta = [bb[D:2 * D]] * 5
                    for j in range(D):
                        wmr = wmb[j, :]
                        wgr = wfb[j, 0:D]
                        wbr = wfb[j, D:2 * D]
                        for ri in range(5):
                            hj = plsc.load_gather(
                                h5, [jnp.full((16,), ri * 16 + j, jnp.int32)])
                            pj[ri] = pj[ri] + hj * wmr
                            fg[ri] = fg[ri] + hj * wgr
                            fbeta[ri] = fbeta[ri] + hj * wbr
                    for ri in range(5):
                        r = r0 + ri
                        hb[r, :] = pj[ri]
                        fob[r, 0:D] = fg[ri]
                        fob[r, D:2 * D] = fbeta[ri]

            out_prev = [pltpu.async_copy(hb, h_out_hbm.at[pl.ds(gbase, CH)],
                                         semo)]
            if matmul:
                out_prev.append(
                    pltpu.async_copy(fob, film_out_hbm.at[pl.ds(gbase, CH)],
                                     semo))
        for dsc in out_prev:
            dsc.wait()

    return k


def kernel(x, edge_index, W_msg, W_film, b_film):
    N, D = x.shape
    E = edge_index.shape[1]
    L = W_msg.shape[0]
    assert D == 16

    per = -(-E // NW)
    n_chunks = -(-per // K)
    per_pad = n_chunks * K
    E_pad = NW * per_pad

    src = edge_index[0]
    dst = edge_index[1]
    if E_pad == E:
        A = -(-N // NS) * NS
    else:
        A = -(-(N + 1) // NS) * NS
        pos = jnp.arange(E_pad, dtype=jnp.int32)
        w = pos // per_pad
        p = pos % per_pad
        g = w * per + p
        valid = (p < per) & (g < E)
        gc = jnp.minimum(g, E - 1)
        src = jnp.where(valid, src[gc], 0)
        dst = jnp.where(valid, dst[gc], N)

    assert N % NW == 0
    edge_k = _make_edge_kernel(A, D, n_chunks)
    deg_k = _make_deg_kernel(A, D, n_chunks)
    mm0 = _make_node_kernel(N, A, D, combine=False, matmul=True)
    mmc = _make_node_kernel(N, A, D, combine=True, matmul=True)
    mmf = _make_node_kernel(N, A, D, combine=True, matmul=False)

    eidx = jnp.stack([src, dst]).reshape(2, NW * n_chunks, K)
    degp = deg_k(dst)
    proj, film = mm0(x, W_msg[0], W_film[0], b_film[0])
    for l in range(L):
        Sp = edge_k(proj, eidx)
        if l < L - 1:
            proj, film = mmc(Sp, degp, film,
                             W_msg[l + 1], W_film[l + 1], b_film[l + 1])
        else:
            h = mmf(Sp, degp, film)
    return h
